# baseline (speedup 1.0000x reference)
import functools

import jax
import jax.numpy as jnp
import numpy as np
from jax import lax
from jax.experimental import pallas as pl
from jax.experimental.pallas import tpu as pltpu
from jax.experimental.pallas import tpu_sc as plsc

N = 10000
E = 320000
D = 128
HID = 128
L = 4
NC = 10
AVG_D_LOG = float(np.log(33.0))

LANES = 16
NV = HID // LANES
NW = 32
HALF = 160
NHALF = 2
NPAD = NW * NHALF * HALF
KB = 96
CE = 84 * KB
SLACK = 32

_neg_inf = float(np.float32(-np.inf))
_pos_inf = float(np.float32(np.inf))


def _sload(ref, i):
    return ref[pl.ds(i, LANES)][0]


def _sc_aggregate(x, src_s, dst_s_pad, rp_pad):
    mesh = plsc.VectorSubcoreMesh(core_axis_name="c", subcore_axis_name="s")

    @functools.partial(
        pl.kernel,
        out_type=jax.ShapeDtypeStruct((4, NPAD * HID), jnp.float32),
        mesh=mesh,
        scratch_types=[
            pltpu.VMEM((HALF + SLACK,), jnp.int32),
            pltpu.VMEM((CE + KB,), jnp.int32),
            pltpu.VMEM((CE + KB,), jnp.int32),
            [pltpu.VMEM((KB, HID), jnp.float32)] * 2,
            [pltpu.VMEM((HALF * HID,), jnp.float32)] * 4,
            [pltpu.SemaphoreType.DMA] * 2,
        ],
    )
    def agg(x_hbm, src_hbm, dst_hbm, rp_hbm, out_hbm, rp_v, sid_v, did_v,
            rows, stg, sems):
        wid = lax.axis_index("s") * 2 + lax.axis_index("c")

        zeros = jnp.zeros((LANES,), jnp.float32)
        ninf = jnp.full((LANES,), _neg_inf, jnp.float32)
        pinf = jnp.full((LANES,), _pos_inf, jnp.float32)

        for half in range(NHALF):
            base = (wid * NHALF + half) * HALF

            pltpu.sync_copy(rp_hbm.at[pl.ds(base, HALF + SLACK)], rp_v)

            def zero_body(i, _):
                for a in range(4):
                    stg[a][pl.ds(i * LANES, LANES)] = zeros
                return 0
            lax.fori_loop(0, HALF * NV, zero_body, 0)

            lo = _sload(rp_v, 0)
            hi = _sload(rp_v, HALF)
            cnt = hi - lo
            nchunk = lax.div(cnt + (CE - 1), CE)

            def flush(prev, s, q, mx, mn):
                sb = (prev - base) * HID
                for k in range(NV):
                    sl = pl.ds(sb + k * LANES, LANES)
                    stg[0][sl] = s[k]
                    stg[1][sl] = mx[k]
                    stg[2][sl] = mn[k]
                    stg[3][sl] = q[k]

            def gather(b, k):
                pltpu.async_copy(
                    x_hbm.at[sid_v.at[pl.ds(b * KB, KB)]], rows[k], sems[k])

            def chunk_body(c, carry):
                c0 = lo + c * CE
                c0a = (c0 // 8) * 8
                coff = c0 - c0a
                pltpu.sync_copy(src_hbm.at[pl.ds(c0a, CE + KB)], sid_v)
                pltpu.sync_copy(dst_hbm.at[pl.ds(c0a, CE + KB)], did_v)
                mc = jnp.minimum(cnt - c * CE, CE)
                nbc = lax.div(coff + mc + (KB - 1), KB)

                @pl.when(nbc > 0)
                def _():
                    gather(0, 0)

                def proc(b, k, carry):
                    @pl.when(b < nbc)
                    def _():
                        pltpu.make_async_copy(
                            x_hbm.at[sid_v.at[pl.ds(0, KB)]],
                            rows[k], sems[k]).wait()

                    lb = jnp.maximum(coff - b * KB, 0)
                    ub = jnp.clip(coff + mc - b * KB, 0, KB)
                    ub = jnp.maximum(ub, lb)

                    def edge_body(j, ec):
                        prev, s, q, mx, mn = ec
                        d = _sload(did_v, b * KB + j)
                        is_new = d != prev

                        @pl.when(is_new & (prev >= 0))
                        def _():
                            flush(prev, s, q, mx, mn)

                        row = [rows[k][j, pl.ds(v * LANES, LANES)]
                               for v in range(NV)]
                        s = tuple(jnp.where(is_new, row[v], s[v] + row[v])
                                  for v in range(NV))
                        q = tuple(
                            jnp.where(is_new, row[v] * row[v],
                                      q[v] + row[v] * row[v])
                            for v in range(NV))
                        mx = tuple(
                            jnp.where(is_new, row[v],
                                      jnp.maximum(mx[v], row[v]))
                            for v in range(NV))
                        mn = tuple(
                            jnp.where(is_new, row[v],
                                      jnp.minimum(mn[v], row[v]))
                            for v in range(NV))
                        return (d, s, q, mx, mn)

                    return lax.fori_loop(lb, ub, edge_body, carry)

                def pair_body(p, carry):
                    b0 = 2 * p

                    @pl.when(b0 + 1 < nbc)
                    def _():
                        gather(b0 + 1, 1)

                    carry = proc(b0, 0, carry)

                    @pl.when(b0 + 2 < nbc)
                    def _():
                        gather(b0 + 2, 0)

                    return proc(b0 + 1, 1, carry)

                npair = lax.div(nbc + 1, 2)
                return lax.fori_loop(0, npair, pair_body, carry)

            init = (jnp.int32(-1), (zeros,) * NV, (zeros,) * NV,
                    (ninf,) * NV, (pinf,) * NV)
            prev, s, q, mx, mn = lax.fori_loop(0, nchunk, chunk_body, init)

            @pl.when(prev >= 0)
            def _():
                flush(prev, s, q, mx, mn)

            for a in range(4):
                pltpu.sync_copy(
                    stg[a], out_hbm.at[a, pl.ds(base * HID, HALF * HID)])

    out = agg(x, src_s, dst_s_pad, rp_pad)
    return out.reshape(4, NPAD, HID)


BM = 400
NB = N // BM


def _enc_body(h_ref, w_ref, b_ref, o_ref):
    o_ref[...] = jnp.dot(h_ref[...], w_ref[...],
                         preferred_element_type=jnp.float32) + b_ref[...]


def _encode(h, enc_W, enc_b):
    return pl.pallas_call(
        _enc_body,
        grid=(NB,),
        in_specs=[
            pl.BlockSpec((BM, D), lambda i: (i, 0)),
            pl.BlockSpec((D, HID), lambda i: (0, 0)),
            pl.BlockSpec((1, HID), lambda i: (0, 0)),
        ],
        out_specs=pl.BlockSpec((BM, HID), lambda i: (i, 0)),
        out_shape=jax.ShapeDtypeStruct((N, HID), jnp.float32),
    )(h, enc_W, enc_b.reshape(1, HID))


def _layer_mm_body(x_ref, sum_ref, mx_ref, mn_ref, sq_ref, deg_ref, w_ref,
                   b_ref, h1_ref, st_ref):
    i = pl.program_id(0)
    deg = deg_ref[...]
    mean = sum_ref[...] / deg
    sq = sq_ref[...] / deg
    std = jnp.sqrt(jax.nn.relu(sq - mean * mean) + 1e-5)
    delta = jnp.maximum(jnp.log(deg + 1.0) / AVG_D_LOG, 1e-5)
    inv = 1.0 / delta
    parts = [x_ref[...], mean, mx_ref[...], mn_ref[...], std,
             mean * delta, mx_ref[...] * delta, mn_ref[...] * delta,
             std * delta, mean * inv, mx_ref[...] * inv, mn_ref[...] * inv,
             std * inv]
    acc = b_ref[...]
    for k, p in enumerate(parts):
        acc = acc + jnp.dot(p, w_ref[k * HID:(k + 1) * HID, :],
                            preferred_element_type=jnp.float32)
    h1 = jax.nn.relu(acc)
    h1_ref[...] = h1

    @pl.when(i == 0)
    def _():
        st_ref[...] = jnp.zeros_like(st_ref)

    st_ref[0:1, :] += jnp.sum(h1, axis=0, keepdims=True)
    st_ref[1:2, :] += jnp.sum(h1 * h1, axis=0, keepdims=True)


def _layer_mm(x, ssum, smax, smin, ssq, deg_col, W, b):
    return pl.pallas_call(
        _layer_mm_body,
        grid=(NB,),
        in_specs=[
            pl.BlockSpec((BM, HID), lambda i: (i, 0)),
            pl.BlockSpec((BM, HID), lambda i: (i, 0)),
            pl.BlockSpec((BM, HID), lambda i: (i, 0)),
            pl.BlockSpec((BM, HID), lambda i: (i, 0)),
            pl.BlockSpec((BM, HID), lambda i: (i, 0)),
            pl.BlockSpec((BM, 1), lambda i: (i, 0)),
            pl.BlockSpec((13 * HID, HID), lambda i: (0, 0)),
            pl.BlockSpec((1, HID), lambda i: (0, 0)),
        ],
        out_specs=[
            pl.BlockSpec((BM, HID), lambda i: (i, 0)),
            pl.BlockSpec((8, HID), lambda i: (0, 0)),
        ],
        out_shape=[
            jax.ShapeDtypeStruct((N, HID), jnp.float32),
            jax.ShapeDtypeStruct((8, HID), jnp.float32),
        ],
    )(x, ssum, smax, smin, ssq, deg_col, W, b.reshape(1, HID))


def _bn_apply_body(h1_ref, x_ref, sc_ref, sh_ref, o_ref, st_ref):
    i = pl.program_id(0)
    out = h1_ref[...] * sc_ref[...] + sh_ref[...] + x_ref[...]
    o_ref[...] = out

    @pl.when(i == 0)
    def _():
        st_ref[...] = jnp.zeros_like(st_ref)

    st_ref[0:1, :] += jnp.sum(out, axis=0, keepdims=True)


def _bn_apply(h1, x, scale, shift):
    return pl.pallas_call(
        _bn_apply_body,
        grid=(NB,),
        in_specs=[
            pl.BlockSpec((BM, HID), lambda i: (i, 0)),
            pl.BlockSpec((BM, HID), lambda i: (i, 0)),
            pl.BlockSpec((1, HID), lambda i: (0, 0)),
            pl.BlockSpec((1, HID), lambda i: (0, 0)),
        ],
        out_specs=[
            pl.BlockSpec((BM, HID), lambda i: (i, 0)),
            pl.BlockSpec((8, HID), lambda i: (0, 0)),
        ],
        out_shape=[
            jax.ShapeDtypeStruct((N, HID), jnp.float32),
            jax.ShapeDtypeStruct((8, HID), jnp.float32),
        ],
    )(h1, x, scale, shift)


def _readout_body(hg_ref, w1_ref, b1_ref, w2_ref, b2_ref, w3_ref, b3_ref,
                  o_ref):
    z = jax.nn.relu(jnp.dot(hg_ref[...], w1_ref[...],
                            preferred_element_type=jnp.float32) + b1_ref[...])
    z = jax.nn.relu(jnp.dot(z, w2_ref[...],
                            preferred_element_type=jnp.float32) + b2_ref[...])
    o_ref[...] = jnp.dot(z, w3_ref[...],
                         preferred_element_type=jnp.float32) + b3_ref[...]


def _readout(hg, r1_W, r1_b, r2_W, r2_b, r3_W, r3_b):
    return pl.pallas_call(
        _readout_body,
        out_shape=jax.ShapeDtypeStruct((1, NC), jnp.float32),
    )(hg, r1_W, r1_b.reshape(1, -1), r2_W, r2_b.reshape(1, -1),
      r3_W, r3_b.reshape(1, -1))


def kernel(h, e, edge_index, enc_W, enc_b, post_W, post_b, bn_g, bn_b,
           r1_W, r1_b, r2_W, r2_b, r3_W, r3_b):
    src = edge_index[0]
    dst = edge_index[1]
    dst_s, src_s = lax.sort((dst, src), num_keys=1)
    src_pad = jnp.concatenate(
        [src_s, jnp.zeros((CE + KB,), jnp.int32)])
    dst_pad = jnp.concatenate(
        [dst_s, jnp.full((CE + KB,), N, jnp.int32)])
    rp = jnp.searchsorted(dst_s, jnp.arange(NPAD + 1, dtype=jnp.int32)
                          ).astype(jnp.int32)
    rp_pad = jnp.concatenate([rp, jnp.full((SLACK,), E, jnp.int32)])

    deg_col = jnp.maximum(
        (rp[1:N + 1] - rp[:N]).astype(jnp.float32), 1.0)[:, None]

    x = _encode(h, enc_W, enc_b)
    for l in range(L):
        aggs = _sc_aggregate(x, src_pad, dst_pad, rp_pad)
        h1, st = _layer_mm(x, aggs[0, :N], aggs[1, :N], aggs[2, :N],
                           aggs[3, :N], deg_col, post_W[l], post_b[l])
        mu = st[0:1, :] / N
        var = st[1:2, :] / N - mu * mu
        scale = bn_g[l][None, :] / jnp.sqrt(var + 1e-5)
        shift = bn_b[l][None, :] - mu * scale
        x, xs = _bn_apply(h1, x, scale, shift)
    hg = xs[0:1, :] / N
    return _readout(hg, r1_W, r1_b, r2_W, r2_b, r3_W, r3_b)


# speedup vs baseline: 1.0365x; 1.0365x over previous
import functools

import jax
import jax.numpy as jnp
import numpy as np
from jax import lax
from jax.experimental import pallas as pl
from jax.experimental.pallas import tpu as pltpu
from jax.experimental.pallas import tpu_sc as plsc

N = 10000
E = 320000
D = 128
HID = 128
L = 4
NC = 10
AVG_D_LOG = float(np.log(33.0))

LANES = 16
NV = HID // LANES
NW = 32
HALF = 160
NHALF = 2
NPAD = NW * NHALF * HALF
KB = 96
CE = 84 * KB
SLACK = 32

_neg_inf = float(np.float32(-np.inf))
_pos_inf = float(np.float32(np.inf))


def _sload(ref, i):
    return ref[pl.ds(i, LANES)][0]


def _sc_aggregate(x, src_s, dst_s_pad, rp_pad):
    mesh = plsc.VectorSubcoreMesh(core_axis_name="c", subcore_axis_name="s")

    @functools.partial(
        pl.kernel,
        out_type=jax.ShapeDtypeStruct((4, NPAD * HID), jnp.float32),
        mesh=mesh,
        scratch_types=[
            pltpu.VMEM((HALF + SLACK,), jnp.int32),
            pltpu.VMEM((CE + KB,), jnp.int32),
            pltpu.VMEM((CE + KB,), jnp.int32),
            [pltpu.VMEM((KB, HID), jnp.float32)] * 2,
            [pltpu.VMEM((HALF * HID,), jnp.float32)] * 4,
            [pltpu.SemaphoreType.DMA] * 2,
        ],
    )
    def agg(x_hbm, src_hbm, dst_hbm, rp_hbm, out_hbm, rp_v, sid_v, did_v,
            rows, stg, sems):
        wid = lax.axis_index("s") * 2 + lax.axis_index("c")

        zeros = jnp.zeros((LANES,), jnp.float32)
        ninf = jnp.full((LANES,), _neg_inf, jnp.float32)
        pinf = jnp.full((LANES,), _pos_inf, jnp.float32)

        for half in range(NHALF):
            base = (wid * NHALF + half) * HALF

            pltpu.sync_copy(rp_hbm.at[pl.ds(base, HALF + SLACK)], rp_v)

            def zero_body(i, _):
                for a in range(4):
                    stg[a][pl.ds(i * LANES, LANES)] = zeros
                return 0
            lax.fori_loop(0, HALF * NV, zero_body, 0)

            lo = _sload(rp_v, 0)
            hi = _sload(rp_v, HALF)
            cnt = hi - lo
            nchunk = lax.div(cnt + (CE - 1), CE)

            def flush(prev, s, q, mx, mn):
                sb = (prev - base) * HID
                for k in range(NV):
                    sl = pl.ds(sb + k * LANES, LANES)
                    stg[0][sl] = s[k]
                    stg[1][sl] = mx[k]
                    stg[2][sl] = mn[k]
                    stg[3][sl] = q[k]

            def gather(b, k):
                pltpu.async_copy(
                    x_hbm.at[sid_v.at[pl.ds(b * KB, KB)]], rows[k], sems[k])

            def chunk_body(c, carry):
                c0 = lo + c * CE
                c0a = (c0 // 8) * 8
                coff = c0 - c0a
                pltpu.sync_copy(src_hbm.at[pl.ds(c0a, CE + KB)], sid_v)
                pltpu.sync_copy(dst_hbm.at[pl.ds(c0a, CE + KB)], did_v)
                mc = jnp.minimum(cnt - c * CE, CE)
                nbc = lax.div(coff + mc + (KB - 1), KB)

                @pl.when(nbc > 0)
                def _():
                    gather(0, 0)

                def proc(b, k, carry):
                    @pl.when(b < nbc)
                    def _():
                        pltpu.make_async_copy(
                            x_hbm.at[sid_v.at[pl.ds(0, KB)]],
                            rows[k], sems[k]).wait()

                    lb = jnp.maximum(coff - b * KB, 0)
                    ub = jnp.clip(coff + mc - b * KB, 0, KB)
                    ub = jnp.maximum(ub, lb)

                    def edge_body(j, ec):
                        prev = ec[0]
                        d = _sload(did_v, b * KB + j)

                        def new_node(cc):
                            prevc, s, q, mx, mn = cc

                            @pl.when(prevc >= 0)
                            def _():
                                flush(prevc, s, q, mx, mn)

                            return (d, (zeros,) * NV, (zeros,) * NV,
                                    (ninf,) * NV, (pinf,) * NV)

                        def same_node(cc):
                            return cc

                        prev, s, q, mx, mn = lax.cond(
                            d != prev, new_node, same_node, ec)

                        row = [rows[k][j, pl.ds(v * LANES, LANES)]
                               for v in range(NV)]
                        s = tuple(s[v] + row[v] for v in range(NV))
                        q = tuple(q[v] + row[v] * row[v] for v in range(NV))
                        mx = tuple(jnp.maximum(mx[v], row[v])
                                   for v in range(NV))
                        mn = tuple(jnp.minimum(mn[v], row[v])
                                   for v in range(NV))
                        return (prev, s, q, mx, mn)

                    return lax.fori_loop(lb, ub, edge_body, carry)

                def pair_body(p, carry):
                    b0 = 2 * p

                    @pl.when(b0 + 1 < nbc)
                    def _():
                        gather(b0 + 1, 1)

                    carry = proc(b0, 0, carry)

                    @pl.when(b0 + 2 < nbc)
                    def _():
                        gather(b0 + 2, 0)

                    return proc(b0 + 1, 1, carry)

                npair = lax.div(nbc + 1, 2)
                return lax.fori_loop(0, npair, pair_body, carry)

            init = (jnp.int32(-1), (zeros,) * NV, (zeros,) * NV,
                    (ninf,) * NV, (pinf,) * NV)
            prev, s, q, mx, mn = lax.fori_loop(0, nchunk, chunk_body, init)

            @pl.when(prev >= 0)
            def _():
                flush(prev, s, q, mx, mn)

            for a in range(4):
                pltpu.sync_copy(
                    stg[a], out_hbm.at[a, pl.ds(base * HID, HALF * HID)])

    out = agg(x, src_s, dst_s_pad, rp_pad)
    return out.reshape(4, NPAD, HID)


BM = 400
NB = N // BM


def _enc_body(h_ref, w_ref, b_ref, o_ref):
    o_ref[...] = jnp.dot(h_ref[...], w_ref[...],
                         preferred_element_type=jnp.float32) + b_ref[...]


def _encode(h, enc_W, enc_b):
    return pl.pallas_call(
        _enc_body,
        grid=(NB,),
        in_specs=[
            pl.BlockSpec((BM, D), lambda i: (i, 0)),
            pl.BlockSpec((D, HID), lambda i: (0, 0)),
            pl.BlockSpec((1, HID), lambda i: (0, 0)),
        ],
        out_specs=pl.BlockSpec((BM, HID), lambda i: (i, 0)),
        out_shape=jax.ShapeDtypeStruct((N, HID), jnp.float32),
    )(h, enc_W, enc_b.reshape(1, HID))


def _layer_mm_body(x_ref, sum_ref, mx_ref, mn_ref, sq_ref, deg_ref, w_ref,
                   b_ref, h1_ref, st_ref):
    i = pl.program_id(0)
    deg = deg_ref[...]
    mean = sum_ref[...] / deg
    sq = sq_ref[...] / deg
    std = jnp.sqrt(jax.nn.relu(sq - mean * mean) + 1e-5)
    delta = jnp.maximum(jnp.log(deg + 1.0) / AVG_D_LOG, 1e-5)
    inv = 1.0 / delta
    parts = [x_ref[...], mean, mx_ref[...], mn_ref[...], std,
             mean * delta, mx_ref[...] * delta, mn_ref[...] * delta,
             std * delta, mean * inv, mx_ref[...] * inv, mn_ref[...] * inv,
             std * inv]
    acc = b_ref[...]
    for k, p in enumerate(parts):
        acc = acc + jnp.dot(p, w_ref[k * HID:(k + 1) * HID, :],
                            preferred_element_type=jnp.float32)
    h1 = jax.nn.relu(acc)
    h1_ref[...] = h1

    @pl.when(i == 0)
    def _():
        st_ref[...] = jnp.zeros_like(st_ref)

    st_ref[0:1, :] += jnp.sum(h1, axis=0, keepdims=True)
    st_ref[1:2, :] += jnp.sum(h1 * h1, axis=0, keepdims=True)


def _layer_mm(x, ssum, smax, smin, ssq, deg_col, W, b):
    return pl.pallas_call(
        _layer_mm_body,
        grid=(NB,),
        in_specs=[
            pl.BlockSpec((BM, HID), lambda i: (i, 0)),
            pl.BlockSpec((BM, HID), lambda i: (i, 0)),
            pl.BlockSpec((BM, HID), lambda i: (i, 0)),
            pl.BlockSpec((BM, HID), lambda i: (i, 0)),
            pl.BlockSpec((BM, HID), lambda i: (i, 0)),
            pl.BlockSpec((BM, 1), lambda i: (i, 0)),
            pl.BlockSpec((13 * HID, HID), lambda i: (0, 0)),
            pl.BlockSpec((1, HID), lambda i: (0, 0)),
        ],
        out_specs=[
            pl.BlockSpec((BM, HID), lambda i: (i, 0)),
            pl.BlockSpec((8, HID), lambda i: (0, 0)),
        ],
        out_shape=[
            jax.ShapeDtypeStruct((N, HID), jnp.float32),
            jax.ShapeDtypeStruct((8, HID), jnp.float32),
        ],
    )(x, ssum, smax, smin, ssq, deg_col, W, b.reshape(1, HID))


def _bn_apply_body(h1_ref, x_ref, sc_ref, sh_ref, o_ref, st_ref):
    i = pl.program_id(0)
    out = h1_ref[...] * sc_ref[...] + sh_ref[...] + x_ref[...]
    o_ref[...] = out

    @pl.when(i == 0)
    def _():
        st_ref[...] = jnp.zeros_like(st_ref)

    st_ref[0:1, :] += jnp.sum(out, axis=0, keepdims=True)


def _bn_apply(h1, x, scale, shift):
    return pl.pallas_call(
        _bn_apply_body,
        grid=(NB,),
        in_specs=[
            pl.BlockSpec((BM, HID), lambda i: (i, 0)),
            pl.BlockSpec((BM, HID), lambda i: (i, 0)),
            pl.BlockSpec((1, HID), lambda i: (0, 0)),
            pl.BlockSpec((1, HID), lambda i: (0, 0)),
        ],
        out_specs=[
            pl.BlockSpec((BM, HID), lambda i: (i, 0)),
            pl.BlockSpec((8, HID), lambda i: (0, 0)),
        ],
        out_shape=[
            jax.ShapeDtypeStruct((N, HID), jnp.float32),
            jax.ShapeDtypeStruct((8, HID), jnp.float32),
        ],
    )(h1, x, scale, shift)


def _readout_body(hg_ref, w1_ref, b1_ref, w2_ref, b2_ref, w3_ref, b3_ref,
                  o_ref):
    z = jax.nn.relu(jnp.dot(hg_ref[...], w1_ref[...],
                            preferred_element_type=jnp.float32) + b1_ref[...])
    z = jax.nn.relu(jnp.dot(z, w2_ref[...],
                            preferred_element_type=jnp.float32) + b2_ref[...])
    o_ref[...] = jnp.dot(z, w3_ref[...],
                         preferred_element_type=jnp.float32) + b3_ref[...]


def _readout(hg, r1_W, r1_b, r2_W, r2_b, r3_W, r3_b):
    return pl.pallas_call(
        _readout_body,
        out_shape=jax.ShapeDtypeStruct((1, NC), jnp.float32),
    )(hg, r1_W, r1_b.reshape(1, -1), r2_W, r2_b.reshape(1, -1),
      r3_W, r3_b.reshape(1, -1))


def kernel(h, e, edge_index, enc_W, enc_b, post_W, post_b, bn_g, bn_b,
           r1_W, r1_b, r2_W, r2_b, r3_W, r3_b):
    src = edge_index[0]
    dst = edge_index[1]
    packed = lax.sort(dst * 16384 + src)
    dst_s = packed >> 14
    src_s = packed & 16383
    src_pad = jnp.concatenate(
        [src_s, jnp.zeros((CE + KB,), jnp.int32)])
    dst_pad = jnp.concatenate(
        [dst_s, jnp.full((CE + KB,), N, jnp.int32)])
    rp = jnp.searchsorted(dst_s, jnp.arange(NPAD + 1, dtype=jnp.int32)
                          ).astype(jnp.int32)
    rp_pad = jnp.concatenate([rp, jnp.full((SLACK,), E, jnp.int32)])

    deg_col = jnp.maximum(
        (rp[1:N + 1] - rp[:N]).astype(jnp.float32), 1.0)[:, None]

    x = _encode(h, enc_W, enc_b)
    for l in range(L):
        aggs = _sc_aggregate(x, src_pad, dst_pad, rp_pad)
        h1, st = _layer_mm(x, aggs[0, :N], aggs[1, :N], aggs[2, :N],
                           aggs[3, :N], deg_col, post_W[l], post_b[l])
        mu = st[0:1, :] / N
        var = st[1:2, :] / N - mu * mu
        scale = bn_g[l][None, :] / jnp.sqrt(var + 1e-5)
        shift = bn_b[l][None, :] - mu * scale
        x, xs = _bn_apply(h1, x, scale, shift)
    hg = xs[0:1, :] / N
    return _readout(hg, r1_W, r1_b, r2_W, r2_b, r3_W, r3_b)


# speedup vs baseline: 1.0366x; 1.0001x over previous
import functools

import jax
import jax.numpy as jnp
import numpy as np
from jax import lax
from jax.experimental import pallas as pl
from jax.experimental.pallas import tpu as pltpu
from jax.experimental.pallas import tpu_sc as plsc

N = 10000
E = 320000
D = 128
HID = 128
L = 4
NC = 10
AVG_D_LOG = float(np.log(33.0))

LANES = 16
NV = HID // LANES
NW = 32
HALF = 160
NHALF = 2
NPAD = NW * NHALF * HALF
KB = 128
CE = 56 * KB
SLACK = 32

_neg_inf = float(np.float32(-np.inf))
_pos_inf = float(np.float32(np.inf))


def _sload(ref, i):
    return ref[pl.ds(i, LANES)][0]


def _sc_aggregate(x, src_s, dst_s_pad, rp_pad):
    mesh = plsc.VectorSubcoreMesh(core_axis_name="c", subcore_axis_name="s")

    @functools.partial(
        pl.kernel,
        out_type=jax.ShapeDtypeStruct((4, NPAD * HID), jnp.float32),
        mesh=mesh,
        scratch_types=[
            pltpu.VMEM((HALF + SLACK,), jnp.int32),
            pltpu.VMEM((CE + KB,), jnp.int32),
            pltpu.VMEM((CE + KB,), jnp.int32),
            [pltpu.VMEM((KB, HID), jnp.float32)] * 2,
            [pltpu.VMEM((HALF * HID,), jnp.float32)] * 4,
            [pltpu.SemaphoreType.DMA] * 2,
        ],
    )
    def agg(x_hbm, src_hbm, dst_hbm, rp_hbm, out_hbm, rp_v, sid_v, did_v,
            rows, stg, sems):
        wid = lax.axis_index("s") * 2 + lax.axis_index("c")

        zeros = jnp.zeros((LANES,), jnp.float32)
        ninf = jnp.full((LANES,), _neg_inf, jnp.float32)
        pinf = jnp.full((LANES,), _pos_inf, jnp.float32)

        for half in range(NHALF):
            base = (wid * NHALF + half) * HALF

            pltpu.sync_copy(rp_hbm.at[pl.ds(base, HALF + SLACK)], rp_v)

            def zero_body(i, _):
                for a in range(4):
                    stg[a][pl.ds(i * LANES, LANES)] = zeros
                return 0
            lax.fori_loop(0, HALF * NV, zero_body, 0)

            lo = _sload(rp_v, 0)
            hi = _sload(rp_v, HALF)
            cnt = hi - lo
            nchunk = lax.div(cnt + (CE - 1), CE)

            def flush(prev, s, q, mx, mn):
                sb = (prev - base) * HID
                for k in range(NV):
                    sl = pl.ds(sb + k * LANES, LANES)
                    stg[0][sl] = s[k]
                    stg[1][sl] = mx[k]
                    stg[2][sl] = mn[k]
                    stg[3][sl] = q[k]

            def gather(b, k):
                pltpu.async_copy(
                    x_hbm.at[sid_v.at[pl.ds(b * KB, KB)]], rows[k], sems[k])

            def chunk_body(c, carry):
                c0 = lo + c * CE
                c0a = (c0 // 8) * 8
                coff = c0 - c0a
                pltpu.sync_copy(src_hbm.at[pl.ds(c0a, CE + KB)], sid_v)
                pltpu.sync_copy(dst_hbm.at[pl.ds(c0a, CE + KB)], did_v)
                mc = jnp.minimum(cnt - c * CE, CE)
                nbc = lax.div(coff + mc + (KB - 1), KB)

                @pl.when(nbc > 0)
                def _():
                    gather(0, 0)

                def proc(b, k, carry):
                    @pl.when(b < nbc)
                    def _():
                        pltpu.make_async_copy(
                            x_hbm.at[sid_v.at[pl.ds(0, KB)]],
                            rows[k], sems[k]).wait()

                    lb = jnp.maximum(coff - b * KB, 0)
                    ub = jnp.clip(coff + mc - b * KB, 0, KB)
                    ub = jnp.maximum(ub, lb)

                    def edge_body(j, ec):
                        prev = ec[0]
                        d = _sload(did_v, b * KB + j)

                        def new_node(cc):
                            prevc, s, q, mx, mn = cc

                            @pl.when(prevc >= 0)
                            def _():
                                flush(prevc, s, q, mx, mn)

                            return (d, (zeros,) * NV, (zeros,) * NV,
                                    (ninf,) * NV, (pinf,) * NV)

                        def same_node(cc):
                            return cc

                        prev, s, q, mx, mn = lax.cond(
                            d != prev, new_node, same_node, ec)

                        row = [rows[k][j, pl.ds(v * LANES, LANES)]
                               for v in range(NV)]
                        s = tuple(s[v] + row[v] for v in range(NV))
                        q = tuple(q[v] + row[v] * row[v] for v in range(NV))
                        mx = tuple(jnp.maximum(mx[v], row[v])
                                   for v in range(NV))
                        mn = tuple(jnp.minimum(mn[v], row[v])
                                   for v in range(NV))
                        return (prev, s, q, mx, mn)

                    return lax.fori_loop(lb, ub, edge_body, carry)

                def pair_body(p, carry):
                    b0 = 2 * p

                    @pl.when(b0 + 1 < nbc)
                    def _():
                        gather(b0 + 1, 1)

                    carry = proc(b0, 0, carry)

                    @pl.when(b0 + 2 < nbc)
                    def _():
                        gather(b0 + 2, 0)

                    return proc(b0 + 1, 1, carry)

                npair = lax.div(nbc + 1, 2)
                return lax.fori_loop(0, npair, pair_body, carry)

            init = (jnp.int32(-1), (zeros,) * NV, (zeros,) * NV,
                    (ninf,) * NV, (pinf,) * NV)
            prev, s, q, mx, mn = lax.fori_loop(0, nchunk, chunk_body, init)

            @pl.when(prev >= 0)
            def _():
                flush(prev, s, q, mx, mn)

            for a in range(4):
                pltpu.sync_copy(
                    stg[a], out_hbm.at[a, pl.ds(base * HID, HALF * HID)])

    out = agg(x, src_s, dst_s_pad, rp_pad)
    return out.reshape(4, NPAD, HID)


BM = 400
NB = N // BM


def _enc_body(h_ref, w_ref, b_ref, o_ref):
    o_ref[...] = jnp.dot(h_ref[...], w_ref[...],
                         preferred_element_type=jnp.float32) + b_ref[...]


def _encode(h, enc_W, enc_b):
    return pl.pallas_call(
        _enc_body,
        grid=(NB,),
        in_specs=[
            pl.BlockSpec((BM, D), lambda i: (i, 0)),
            pl.BlockSpec((D, HID), lambda i: (0, 0)),
            pl.BlockSpec((1, HID), lambda i: (0, 0)),
        ],
        out_specs=pl.BlockSpec((BM, HID), lambda i: (i, 0)),
        out_shape=jax.ShapeDtypeStruct((N, HID), jnp.float32),
    )(h, enc_W, enc_b.reshape(1, HID))


def _layer_mm_body(x_ref, sum_ref, mx_ref, mn_ref, sq_ref, deg_ref, w_ref,
                   b_ref, h1_ref, st_ref):
    i = pl.program_id(0)
    deg = deg_ref[...]
    mean = sum_ref[...] / deg
    sq = sq_ref[...] / deg
    std = jnp.sqrt(jax.nn.relu(sq - mean * mean) + 1e-5)
    delta = jnp.maximum(jnp.log(deg + 1.0) / AVG_D_LOG, 1e-5)
    inv = 1.0 / delta
    parts = [x_ref[...], mean, mx_ref[...], mn_ref[...], std,
             mean * delta, mx_ref[...] * delta, mn_ref[...] * delta,
             std * delta, mean * inv, mx_ref[...] * inv, mn_ref[...] * inv,
             std * inv]
    acc = b_ref[...]
    for k, p in enumerate(parts):
        acc = acc + jnp.dot(p, w_ref[k * HID:(k + 1) * HID, :],
                            preferred_element_type=jnp.float32)
    h1 = jax.nn.relu(acc)
    h1_ref[...] = h1

    @pl.when(i == 0)
    def _():
        st_ref[...] = jnp.zeros_like(st_ref)

    st_ref[0:1, :] += jnp.sum(h1, axis=0, keepdims=True)
    st_ref[1:2, :] += jnp.sum(h1 * h1, axis=0, keepdims=True)


def _layer_mm(x, ssum, smax, smin, ssq, deg_col, W, b):
    return pl.pallas_call(
        _layer_mm_body,
        grid=(NB,),
        in_specs=[
            pl.BlockSpec((BM, HID), lambda i: (i, 0)),
            pl.BlockSpec((BM, HID), lambda i: (i, 0)),
            pl.BlockSpec((BM, HID), lambda i: (i, 0)),
            pl.BlockSpec((BM, HID), lambda i: (i, 0)),
            pl.BlockSpec((BM, HID), lambda i: (i, 0)),
            pl.BlockSpec((BM, 1), lambda i: (i, 0)),
            pl.BlockSpec((13 * HID, HID), lambda i: (0, 0)),
            pl.BlockSpec((1, HID), lambda i: (0, 0)),
        ],
        out_specs=[
            pl.BlockSpec((BM, HID), lambda i: (i, 0)),
            pl.BlockSpec((8, HID), lambda i: (0, 0)),
        ],
        out_shape=[
            jax.ShapeDtypeStruct((N, HID), jnp.float32),
            jax.ShapeDtypeStruct((8, HID), jnp.float32),
        ],
    )(x, ssum, smax, smin, ssq, deg_col, W, b.reshape(1, HID))


def _bn_apply_body(h1_ref, x_ref, sc_ref, sh_ref, o_ref, st_ref):
    i = pl.program_id(0)
    out = h1_ref[...] * sc_ref[...] + sh_ref[...] + x_ref[...]
    o_ref[...] = out

    @pl.when(i == 0)
    def _():
        st_ref[...] = jnp.zeros_like(st_ref)

    st_ref[0:1, :] += jnp.sum(out, axis=0, keepdims=True)


def _bn_apply(h1, x, scale, shift):
    return pl.pallas_call(
        _bn_apply_body,
        grid=(NB,),
        in_specs=[
            pl.BlockSpec((BM, HID), lambda i: (i, 0)),
            pl.BlockSpec((BM, HID), lambda i: (i, 0)),
            pl.BlockSpec((1, HID), lambda i: (0, 0)),
            pl.BlockSpec((1, HID), lambda i: (0, 0)),
        ],
        out_specs=[
            pl.BlockSpec((BM, HID), lambda i: (i, 0)),
            pl.BlockSpec((8, HID), lambda i: (0, 0)),
        ],
        out_shape=[
            jax.ShapeDtypeStruct((N, HID), jnp.float32),
            jax.ShapeDtypeStruct((8, HID), jnp.float32),
        ],
    )(h1, x, scale, shift)


def _readout_body(hg_ref, w1_ref, b1_ref, w2_ref, b2_ref, w3_ref, b3_ref,
                  o_ref):
    z = jax.nn.relu(jnp.dot(hg_ref[...], w1_ref[...],
                            preferred_element_type=jnp.float32) + b1_ref[...])
    z = jax.nn.relu(jnp.dot(z, w2_ref[...],
                            preferred_element_type=jnp.float32) + b2_ref[...])
    o_ref[...] = jnp.dot(z, w3_ref[...],
                         preferred_element_type=jnp.float32) + b3_ref[...]


def _readout(hg, r1_W, r1_b, r2_W, r2_b, r3_W, r3_b):
    return pl.pallas_call(
        _readout_body,
        out_shape=jax.ShapeDtypeStruct((1, NC), jnp.float32),
    )(hg, r1_W, r1_b.reshape(1, -1), r2_W, r2_b.reshape(1, -1),
      r3_W, r3_b.reshape(1, -1))


def kernel(h, e, edge_index, enc_W, enc_b, post_W, post_b, bn_g, bn_b,
           r1_W, r1_b, r2_W, r2_b, r3_W, r3_b):
    src = edge_index[0]
    dst = edge_index[1]
    packed = lax.sort(dst * 16384 + src)
    dst_s = packed >> 14
    src_s = packed & 16383
    src_pad = jnp.concatenate(
        [src_s, jnp.zeros((CE + KB,), jnp.int32)])
    dst_pad = jnp.concatenate(
        [dst_s, jnp.full((CE + KB,), N, jnp.int32)])
    rp = jnp.searchsorted(dst_s, jnp.arange(NPAD + 1, dtype=jnp.int32)
                          ).astype(jnp.int32)
    rp_pad = jnp.concatenate([rp, jnp.full((SLACK,), E, jnp.int32)])

    deg_col = jnp.maximum(
        (rp[1:N + 1] - rp[:N]).astype(jnp.float32), 1.0)[:, None]

    x = _encode(h, enc_W, enc_b)
    for l in range(L):
        aggs = _sc_aggregate(x, src_pad, dst_pad, rp_pad)
        h1, st = _layer_mm(x, aggs[0, :N], aggs[1, :N], aggs[2, :N],
                           aggs[3, :N], deg_col, post_W[l], post_b[l])
        mu = st[0:1, :] / N
        var = st[1:2, :] / N - mu * mu
        scale = bn_g[l][None, :] / jnp.sqrt(var + 1e-5)
        shift = bn_b[l][None, :] - mu * scale
        x, xs = _bn_apply(h1, x, scale, shift)
    hg = xs[0:1, :] / N
    return _readout(hg, r1_W, r1_b, r2_W, r2_b, r3_W, r3_b)


# speedup vs baseline: 1.0368x; 1.0002x over previous
import functools

import jax
import jax.numpy as jnp
import numpy as np
from jax import lax
from jax.experimental import pallas as pl
from jax.experimental.pallas import tpu as pltpu
from jax.experimental.pallas import tpu_sc as plsc

N = 10000
E = 320000
D = 128
HID = 128
L = 4
NC = 10
AVG_D_LOG = float(np.log(33.0))

LANES = 16
NV = HID // LANES
NW = 32
HALF = 160
NHALF = 2
NPAD = NW * NHALF * HALF
KB = 96
CE = 84 * KB
SLACK = 32

_neg_inf = float(np.float32(-np.inf))
_pos_inf = float(np.float32(np.inf))


def _sload(ref, i):
    return ref[pl.ds(i, LANES)][0]


def _sc_aggregate(x, src_s, dst_s_pad, rp_pad):
    mesh = plsc.VectorSubcoreMesh(core_axis_name="c", subcore_axis_name="s")

    @functools.partial(
        pl.kernel,
        out_type=jax.ShapeDtypeStruct((4, NPAD * HID), jnp.float32),
        mesh=mesh,
        scratch_types=[
            pltpu.VMEM((HALF + SLACK,), jnp.int32),
            pltpu.VMEM((CE + KB,), jnp.int32),
            pltpu.VMEM((CE + KB,), jnp.int32),
            [pltpu.VMEM((KB, HID), jnp.float32)] * 2,
            [pltpu.VMEM((HALF * HID,), jnp.float32)] * 4,
            [pltpu.SemaphoreType.DMA] * 2,
        ],
    )
    def agg(x_hbm, src_hbm, dst_hbm, rp_hbm, out_hbm, rp_v, sid_v, did_v,
            rows, stg, sems):
        wid = lax.axis_index("s") * 2 + lax.axis_index("c")

        zeros = jnp.zeros((LANES,), jnp.float32)
        ninf = jnp.full((LANES,), _neg_inf, jnp.float32)
        pinf = jnp.full((LANES,), _pos_inf, jnp.float32)

        for half in range(NHALF):
            base = (wid * NHALF + half) * HALF

            pltpu.sync_copy(rp_hbm.at[pl.ds(base, HALF + SLACK)], rp_v)

            def zero_body(i, _):
                for a in range(4):
                    stg[a][pl.ds(i * LANES, LANES)] = zeros
                return 0
            lax.fori_loop(0, HALF * NV, zero_body, 0)

            lo = _sload(rp_v, 0)
            hi = _sload(rp_v, HALF)
            cnt = hi - lo
            nchunk = lax.div(cnt + (CE - 1), CE)

            def flush(prev, s, q, mx, mn):
                sb = (prev - base) * HID
                for k in range(NV):
                    sl = pl.ds(sb + k * LANES, LANES)
                    stg[0][sl] = s[k]
                    stg[1][sl] = mx[k]
                    stg[2][sl] = mn[k]
                    stg[3][sl] = q[k]

            def gather(b, k):
                pltpu.async_copy(
                    x_hbm.at[sid_v.at[pl.ds(b * KB, KB)]], rows[k], sems[k])

            def chunk_body(c, carry):
                c0 = lo + c * CE
                c0a = (c0 // 8) * 8
                coff = c0 - c0a
                pltpu.sync_copy(src_hbm.at[pl.ds(c0a, CE + KB)], sid_v)
                pltpu.sync_copy(dst_hbm.at[pl.ds(c0a, CE + KB)], did_v)
                mc = jnp.minimum(cnt - c * CE, CE)
                nbc = lax.div(coff + mc + (KB - 1), KB)

                @pl.when(nbc > 0)
                def _():
                    gather(0, 0)

                def proc(b, k, carry):
                    @pl.when(b < nbc)
                    def _():
                        pltpu.make_async_copy(
                            x_hbm.at[sid_v.at[pl.ds(0, KB)]],
                            rows[k], sems[k]).wait()

                    lb = jnp.maximum(coff - b * KB, 0)
                    ub = jnp.clip(coff + mc - b * KB, 0, KB)
                    ub = jnp.maximum(ub, lb)

                    def edge_body(j, ec):
                        prev = ec[0]
                        d = _sload(did_v, b * KB + j)

                        def new_node(cc):
                            prevc, s, q, mx, mn = cc

                            @pl.when(prevc >= 0)
                            def _():
                                flush(prevc, s, q, mx, mn)

                            return (d, (zeros,) * NV, (zeros,) * NV,
                                    (ninf,) * NV, (pinf,) * NV)

                        def same_node(cc):
                            return cc

                        prev, s, q, mx, mn = lax.cond(
                            d != prev, new_node, same_node, ec)

                        row = [rows[k][j, pl.ds(v * LANES, LANES)]
                               for v in range(NV)]
                        s = tuple(s[v] + row[v] for v in range(NV))
                        q = tuple(q[v] + row[v] * row[v] for v in range(NV))
                        mx = tuple(jnp.maximum(mx[v], row[v])
                                   for v in range(NV))
                        mn = tuple(jnp.minimum(mn[v], row[v])
                                   for v in range(NV))
                        return (prev, s, q, mx, mn)

                    return lax.fori_loop(lb, ub, edge_body, carry)

                def pair_body(p, carry):
                    b0 = 2 * p

                    @pl.when(b0 + 1 < nbc)
                    def _():
                        gather(b0 + 1, 1)

                    carry = proc(b0, 0, carry)

                    @pl.when(b0 + 2 < nbc)
                    def _():
                        gather(b0 + 2, 0)

                    return proc(b0 + 1, 1, carry)

                npair = lax.div(nbc + 1, 2)
                return lax.fori_loop(0, npair, pair_body, carry)

            init = (jnp.int32(-1), (zeros,) * NV, (zeros,) * NV,
                    (ninf,) * NV, (pinf,) * NV)
            prev, s, q, mx, mn = lax.fori_loop(0, nchunk, chunk_body, init)

            @pl.when(prev >= 0)
            def _():
                flush(prev, s, q, mx, mn)

            for a in range(4):
                pltpu.sync_copy(
                    stg[a], out_hbm.at[a, pl.ds(base * HID, HALF * HID)])

    out = agg(x, src_s, dst_s_pad, rp_pad)
    return out.reshape(4, NPAD, HID)


BM = 400
NB = N // BM


def _enc_body(h_ref, w_ref, b_ref, o_ref):
    o_ref[...] = jnp.dot(h_ref[...], w_ref[...],
                         preferred_element_type=jnp.float32) + b_ref[...]


def _encode(h, enc_W, enc_b):
    return pl.pallas_call(
        _enc_body,
        grid=(NB,),
        in_specs=[
            pl.BlockSpec((BM, D), lambda i: (i, 0)),
            pl.BlockSpec((D, HID), lambda i: (0, 0)),
            pl.BlockSpec((1, HID), lambda i: (0, 0)),
        ],
        out_specs=pl.BlockSpec((BM, HID), lambda i: (i, 0)),
        out_shape=jax.ShapeDtypeStruct((N, HID), jnp.float32),
    )(h, enc_W, enc_b.reshape(1, HID))


def _layer_mm_body(x_ref, sum_ref, mx_ref, mn_ref, sq_ref, deg_ref, w_ref,
                   b_ref, h1_ref, st_ref):
    i = pl.program_id(0)
    deg = deg_ref[...]
    mean = sum_ref[...] / deg
    sq = sq_ref[...] / deg
    std = jnp.sqrt(jax.nn.relu(sq - mean * mean) + 1e-5)
    delta = jnp.maximum(jnp.log(deg + 1.0) / AVG_D_LOG, 1e-5)
    inv = 1.0 / delta
    parts = [x_ref[...], mean, mx_ref[...], mn_ref[...], std,
             mean * delta, mx_ref[...] * delta, mn_ref[...] * delta,
             std * delta, mean * inv, mx_ref[...] * inv, mn_ref[...] * inv,
             std * inv]
    acc = b_ref[...]
    for k, p in enumerate(parts):
        acc = acc + jnp.dot(p, w_ref[k * HID:(k + 1) * HID, :],
                            preferred_element_type=jnp.float32)
    h1 = jax.nn.relu(acc)
    h1_ref[...] = h1

    @pl.when(i == 0)
    def _():
        st_ref[...] = jnp.zeros_like(st_ref)

    st_ref[0:1, :] += jnp.sum(h1, axis=0, keepdims=True)
    st_ref[1:2, :] += jnp.sum(h1 * h1, axis=0, keepdims=True)


def _layer_mm(x, ssum, smax, smin, ssq, deg_col, W, b):
    return pl.pallas_call(
        _layer_mm_body,
        grid=(NB,),
        in_specs=[
            pl.BlockSpec((BM, HID), lambda i: (i, 0)),
            pl.BlockSpec((BM, HID), lambda i: (i, 0)),
            pl.BlockSpec((BM, HID), lambda i: (i, 0)),
            pl.BlockSpec((BM, HID), lambda i: (i, 0)),
            pl.BlockSpec((BM, HID), lambda i: (i, 0)),
            pl.BlockSpec((BM, 1), lambda i: (i, 0)),
            pl.BlockSpec((13 * HID, HID), lambda i: (0, 0)),
            pl.BlockSpec((1, HID), lambda i: (0, 0)),
        ],
        out_specs=[
            pl.BlockSpec((BM, HID), lambda i: (i, 0)),
            pl.BlockSpec((8, HID), lambda i: (0, 0)),
        ],
        out_shape=[
            jax.ShapeDtypeStruct((N, HID), jnp.float32),
            jax.ShapeDtypeStruct((8, HID), jnp.float32),
        ],
    )(x, ssum, smax, smin, ssq, deg_col, W, b.reshape(1, HID))


def _bn_apply_body(h1_ref, x_ref, sc_ref, sh_ref, o_ref, st_ref):
    i = pl.program_id(0)
    out = h1_ref[...] * sc_ref[...] + sh_ref[...] + x_ref[...]
    o_ref[...] = out

    @pl.when(i == 0)
    def _():
        st_ref[...] = jnp.zeros_like(st_ref)

    st_ref[0:1, :] += jnp.sum(out, axis=0, keepdims=True)


def _bn_apply(h1, x, scale, shift):
    return pl.pallas_call(
        _bn_apply_body,
        grid=(NB,),
        in_specs=[
            pl.BlockSpec((BM, HID), lambda i: (i, 0)),
            pl.BlockSpec((BM, HID), lambda i: (i, 0)),
            pl.BlockSpec((1, HID), lambda i: (0, 0)),
            pl.BlockSpec((1, HID), lambda i: (0, 0)),
        ],
        out_specs=[
            pl.BlockSpec((BM, HID), lambda i: (i, 0)),
            pl.BlockSpec((8, HID), lambda i: (0, 0)),
        ],
        out_shape=[
            jax.ShapeDtypeStruct((N, HID), jnp.float32),
            jax.ShapeDtypeStruct((8, HID), jnp.float32),
        ],
    )(h1, x, scale, shift)


def _readout_body(hg_ref, w1_ref, b1_ref, w2_ref, b2_ref, w3_ref, b3_ref,
                  o_ref):
    z = jax.nn.relu(jnp.dot(hg_ref[...], w1_ref[...],
                            preferred_element_type=jnp.float32) + b1_ref[...])
    z = jax.nn.relu(jnp.dot(z, w2_ref[...],
                            preferred_element_type=jnp.float32) + b2_ref[...])
    o_ref[...] = jnp.dot(z, w3_ref[...],
                         preferred_element_type=jnp.float32) + b3_ref[...]


def _readout(hg, r1_W, r1_b, r2_W, r2_b, r3_W, r3_b):
    return pl.pallas_call(
        _readout_body,
        out_shape=jax.ShapeDtypeStruct((1, NC), jnp.float32),
    )(hg, r1_W, r1_b.reshape(1, -1), r2_W, r2_b.reshape(1, -1),
      r3_W, r3_b.reshape(1, -1))


def kernel(h, e, edge_index, enc_W, enc_b, post_W, post_b, bn_g, bn_b,
           r1_W, r1_b, r2_W, r2_b, r3_W, r3_b):
    src = edge_index[0]
    dst = edge_index[1]
    packed = lax.sort(dst * 16384 + src)
    dst_s = packed >> 14
    src_s = packed & 16383
    src_pad = jnp.concatenate(
        [src_s, jnp.zeros((CE + KB,), jnp.int32)])
    dst_pad = jnp.concatenate(
        [dst_s, jnp.full((CE + KB,), N, jnp.int32)])
    rp = jnp.searchsorted(dst_s, jnp.arange(NPAD + 1, dtype=jnp.int32)
                          ).astype(jnp.int32)
    rp_pad = jnp.concatenate([rp, jnp.full((SLACK,), E, jnp.int32)])

    deg_col = jnp.maximum(
        (rp[1:N + 1] - rp[:N]).astype(jnp.float32), 1.0)[:, None]

    x = _encode(h, enc_W, enc_b)
    for l in range(L):
        aggs = _sc_aggregate(x, src_pad, dst_pad, rp_pad)
        h1, st = _layer_mm(x, aggs[0, :N], aggs[1, :N], aggs[2, :N],
                           aggs[3, :N], deg_col, post_W[l], post_b[l])
        mu = st[0:1, :] / N
        var = st[1:2, :] / N - mu * mu
        scale = bn_g[l][None, :] / jnp.sqrt(var + 1e-5)
        shift = bn_b[l][None, :] - mu * scale
        x, xs = _bn_apply(h1, x, scale, shift)
    hg = xs[0:1, :] / N
    return _readout(hg, r1_W, r1_b, r2_W, r2_b, r3_W, r3_b)
